# baseline (speedup 1.0000x reference)
import functools

import jax
import jax.numpy as jnp
from jax import lax
from jax.experimental import pallas as pl
from jax.experimental.pallas import tpu as pltpu
from jax.experimental.pallas import tpu_sc as plsc

N_NODES = 10000
M = 10240
E_EDGES = 160000
BM = 256
GRID_M = M // BM

F32 = jnp.float32


def _mlp_body(nf, emb, w1a, w1b, b1, w2, b2, o1, o2):
    m = pl.program_id(0)
    h = (jnp.dot(nf[...], w1a[...], preferred_element_type=F32)
         + jnp.dot(emb[...], w1b[...], preferred_element_type=F32) + b1[...])
    h = jnp.maximum(h, 0.0)
    x = jnp.dot(h, w2[...], preferred_element_type=F32) + b2[...]
    rows = m * BM + lax.broadcasted_iota(jnp.int32, (BM, 1), 0)
    maskf = (rows < N_NODES).astype(F32)
    x = x * maskf
    full = jnp.concatenate([x, maskf, jnp.zeros((BM, 31), F32)], axis=1)
    o1[...] = full[:, :144]
    o2[...] = full[:, 144:288]


def _run_mlp(nf_pad, emb_g, w1a, w1b, b1, w2, b2):
    din_a, din_b = nf_pad.shape[1], emb_g.shape[1]
    exp, gin = w1a.shape[1], w2.shape[1]
    return pl.pallas_call(
        _mlp_body,
        grid=(GRID_M,),
        in_specs=[
            pl.BlockSpec((BM, din_a), lambda m: (m, 0)),
            pl.BlockSpec((BM, din_b), lambda m: (m, 0)),
            pl.BlockSpec((din_a, exp), lambda m: (0, 0)),
            pl.BlockSpec((din_b, exp), lambda m: (0, 0)),
            pl.BlockSpec((1, exp), lambda m: (0, 0)),
            pl.BlockSpec((exp, gin), lambda m: (0, 0)),
            pl.BlockSpec((1, gin), lambda m: (0, 0)),
        ],
        out_specs=[
            pl.BlockSpec((BM, 144), lambda m: (m, 0)),
            pl.BlockSpec((BM, 144), lambda m: (m, 0)),
        ],
        out_shape=[
            jax.ShapeDtypeStruct((M, 144), F32),
            jax.ShapeDtypeStruct((M, 144), F32),
        ],
    )(nf_pad, emb_g, w1a, w1b, b1, w2, b2)


def _sage_mid_body(din, cnt_col, pw_out, *refs):
    npan_in = (len(refs) - 6) // 3
    aA = refs[:npan_in]
    aB = refs[npan_in:2 * npan_in]
    xs = refs[2 * npan_in:3 * npan_in]
    lw, rw, lb = refs[3 * npan_in:3 * npan_in + 3]
    outs = refs[3 * npan_in + 3:]
    m = pl.program_id(0)
    a = (jnp.concatenate([r[...] for r in aA], axis=1)
         + jnp.concatenate([r[...] for r in aB], axis=1))
    cnt = a[:, cnt_col:cnt_col + 1]
    inv = 1.0 / jnp.maximum(cnt, 1.0)
    mean = a[:, :din] * inv
    x = jnp.concatenate([r[...] for r in xs], axis=1)[:, :din]
    y = (jnp.dot(mean, lw[...], preferred_element_type=F32)
         + jnp.dot(x, rw[...], preferred_element_type=F32) + lb[...])
    rows = m * BM + lax.broadcasted_iota(jnp.int32, (BM, 1), 0)
    maskf = (rows < N_NODES).astype(F32)
    y = y * maskf
    nrm = jnp.sqrt(jnp.sum(y * y, axis=1, keepdims=True))
    z = y / jnp.maximum(nrm, 1e-12)
    z = jnp.where(z >= 0, z, 0.01 * z)
    full = jnp.concatenate([z, maskf, jnp.zeros((BM, 15), F32)], axis=1)
    for i, o in enumerate(outs):
        o[...] = full[:, i * pw_out:(i + 1) * pw_out]


def _run_sage_mid(aA, aB, xs, lw, rw, lb, din, cnt_col):
    npan_in = len(aA)
    pw_in = aA[0].shape[1]
    gh = lw.shape[1]
    pw_out, npan_out = 176, 3
    body = functools.partial(_sage_mid_body, din, cnt_col, pw_out)
    pan_spec = pl.BlockSpec((BM, pw_in), lambda m: (m, 0))
    return pl.pallas_call(
        body,
        grid=(GRID_M,),
        in_specs=(
            [pan_spec] * (2 * npan_in)
            + [pl.BlockSpec((BM, pw_in), lambda m: (m, 0))] * npan_in
            + [
                pl.BlockSpec((din, gh), lambda m: (0, 0)),
                pl.BlockSpec((din, gh), lambda m: (0, 0)),
                pl.BlockSpec((1, gh), lambda m: (0, 0)),
            ]
        ),
        out_specs=[pl.BlockSpec((BM, pw_out), lambda m: (m, 0))] * npan_out,
        out_shape=[jax.ShapeDtypeStruct((M, pw_out), F32)] * npan_out,
    )(*aA, *aB, *xs, lw, rw, lb)


def _sage_last_body(din, cnt_col, *refs):
    npan_in = (len(refs) - 7) // 3
    aA = refs[:npan_in]
    aB = refs[npan_in:2 * npan_in]
    xs = refs[2 * npan_in:3 * npan_in]
    lw, rw, lb, clsw, sep, prev = refs[3 * npan_in:3 * npan_in + 6]
    out = refs[-1]
    m = pl.program_id(0)
    a = (jnp.concatenate([r[...] for r in aA], axis=1)
         + jnp.concatenate([r[...] for r in aB], axis=1))
    cnt = a[:, cnt_col:cnt_col + 1]
    inv = 1.0 / jnp.maximum(cnt, 1.0)
    mean = a[:, :din] * inv
    x = jnp.concatenate([r[...] for r in xs], axis=1)[:, :din]
    y = (jnp.dot(mean, lw[...], preferred_element_type=F32)
         + jnp.dot(x, rw[...], preferred_element_type=F32) + lb[...])
    rows = m * BM + lax.broadcasted_iota(jnp.int32, (BM, 1), 0)
    maskf = (rows < N_NODES).astype(F32)
    y = y * maskf
    ycls = jnp.dot(y, clsw[...], preferred_element_type=F32)
    rowv = m * BM + lax.broadcasted_iota(jnp.int32, (16, BM), 1)
    s = sep[...][:, 0:1]
    p = prev[...][:, 0:1]
    ind = ((rowv >= p) & (rowv < s)).astype(F32)
    contrib = jnp.dot(ind, ycls, preferred_element_type=F32)

    @pl.when(m == 0)
    def _():
        out[...] = jnp.zeros_like(out)

    out[...] += contrib


def _run_sage_last(aA, aB, xs, lw, rw, lb, cls_tiled, sep128, prev128, din, cnt_col):
    npan_in = len(aA)
    pw_in = aA[0].shape[1]
    gh = lw.shape[1]
    body = functools.partial(_sage_last_body, din, cnt_col)
    pan_spec = pl.BlockSpec((BM, pw_in), lambda m: (m, 0))
    return pl.pallas_call(
        body,
        grid=(GRID_M,),
        in_specs=(
            [pan_spec] * (3 * npan_in)
            + [
                pl.BlockSpec((din, gh), lambda m: (0, 0)),
                pl.BlockSpec((din, gh), lambda m: (0, 0)),
                pl.BlockSpec((1, gh), lambda m: (0, 0)),
                pl.BlockSpec((gh, 128), lambda m: (0, 0)),
                pl.BlockSpec((16, 128), lambda m: (0, 0)),
                pl.BlockSpec((16, 128), lambda m: (0, 0)),
            ]
        ),
        out_specs=pl.BlockSpec((16, 128), lambda m: (0, 0)),
        out_shape=jax.ShapeDtypeStruct((16, 128), F32),
    )(*aA, *aB, *xs, lw, rw, lb, cls_tiled, sep128, prev128)


def _embed_gather(opcode_emb, ops_pad):
    return opcode_emb[ops_pad]


def _aggregate(x_panels, src, dst, pw):
    xfull = jnp.concatenate(x_panels, axis=1)
    agg = jax.ops.segment_sum(xfull[src], dst, num_segments=M)
    npan = len(x_panels)
    aA = [agg[:, i * pw:(i + 1) * pw] for i in range(npan)]
    aB = [jnp.zeros_like(a) for a in aA]
    return aA, aB


def kernel(node_features, node_separation, node_ops, edges, batches, opcode_emb,
           mlp_W1, mlp_b1, mlp_W2, mlp_b2,
           s0_lW, s0_lb, s0_rW, s1_lW, s1_lb, s1_rW, s2_lW, s2_lb, s2_rW,
           cls_W, cls_b):
    pad = M - N_NODES
    nf_pad = jnp.pad(node_features, ((0, pad), (0, 0)))
    ops_pad = jnp.pad(node_ops, (0, pad))
    src, dst = edges[0], edges[1]

    emb_g = _embed_gather(opcode_emb, ops_pad)

    w1a, w1b = mlp_W1[:126], mlp_W1[126:]
    x0p = _run_mlp(nf_pad, emb_g, w1a, w1b, mlp_b1.reshape(1, -1),
                   mlp_W2, mlp_b2.reshape(1, -1))

    aA, aB = _aggregate(x0p, src, dst, 144)
    x1p = _run_sage_mid(aA, aB, x0p, s0_lW, s0_rW, s0_lb.reshape(1, -1),
                        din=256, cnt_col=256)
    aA, aB = _aggregate(x1p, src, dst, 176)
    x2p = _run_sage_mid(aA, aB, x1p, s1_lW, s1_rW, s1_lb.reshape(1, -1),
                        din=512, cnt_col=512)
    aA, aB = _aggregate(x2p, src, dst, 176)
    sep128 = jnp.broadcast_to(node_separation.reshape(16, 1), (16, 128))
    prev = jnp.concatenate([jnp.zeros((1,), node_separation.dtype),
                            node_separation[:15]])
    prev128 = jnp.broadcast_to(prev.reshape(16, 1), (16, 128))
    cls_tiled = jnp.broadcast_to(cls_W, (cls_W.shape[0], 128))
    out128 = _run_sage_last(aA, aB, x2p, s2_lW, s2_rW, s2_lb.reshape(1, -1),
                            cls_tiled, sep128, prev128, din=512, cnt_col=512)

    t = out128[:, 0]
    return jnp.zeros((16,), F32).at[batches].set(t) + cls_b


# speedup vs baseline: 3.1821x; 3.1821x over previous
import functools

import jax
import jax.numpy as jnp
from jax import lax
from jax.experimental import pallas as pl
from jax.experimental.pallas import tpu as pltpu
from jax.experimental.pallas import tpu_sc as plsc

N_NODES = 10000
M = 10240
E_EDGES = 160000
BM = 256
GRID_M = M // BM
PW = 128

F32 = jnp.float32
BF16 = jnp.bfloat16


def _bdot(a, b):
    return jnp.dot(a.astype(BF16), b.astype(BF16), preferred_element_type=F32)


def _mlp_body(nf, emb, w1a, w1b, b1, w2, b2, o1, o2):
    m = pl.program_id(0)
    h = _bdot(nf[...], w1a[...]) + _bdot(emb[...][:, :64], w1b[...]) + b1[...]
    h = jnp.maximum(h, 0.0)
    x = _bdot(h, w2[...]) + b2[...]
    rows = m * BM + lax.broadcasted_iota(jnp.int32, (BM, 1), 0)
    maskf = (rows < N_NODES).astype(F32)
    x = x * maskf
    o1[...] = x[:, :PW]
    o2[...] = x[:, PW:2 * PW]


def _run_mlp(nf_pad, emb_g, w1a, w1b, b1, w2, b2):
    din_a = nf_pad.shape[1]
    exp, gin = w1a.shape[1], w2.shape[1]
    return pl.pallas_call(
        _mlp_body,
        grid=(GRID_M,),
        in_specs=[
            pl.BlockSpec((BM, din_a), lambda m: (m, 0)),
            pl.BlockSpec((BM, 128), lambda m: (m, 0)),
            pl.BlockSpec((din_a, exp), lambda m: (0, 0)),
            pl.BlockSpec((64, exp), lambda m: (0, 0)),
            pl.BlockSpec((1, exp), lambda m: (0, 0)),
            pl.BlockSpec((exp, gin), lambda m: (0, 0)),
            pl.BlockSpec((1, gin), lambda m: (0, 0)),
        ],
        out_specs=[
            pl.BlockSpec((BM, PW), lambda m: (m, 0)),
            pl.BlockSpec((BM, PW), lambda m: (m, 0)),
        ],
        out_shape=[
            jax.ShapeDtypeStruct((M, PW), F32),
            jax.ShapeDtypeStruct((M, PW), F32),
        ],
    )(nf_pad, emb_g, w1a, w1b, b1, w2, b2)


def _sage_mid_body(din, *refs):
    npan_in = (len(refs) - 8) // 3
    aA = refs[:npan_in]
    aB = refs[npan_in:2 * npan_in]
    xs = refs[2 * npan_in:3 * npan_in]
    inv, lw, rw, lb = refs[3 * npan_in:3 * npan_in + 4]
    outs = refs[3 * npan_in + 4:]
    m = pl.program_id(0)
    a = (jnp.concatenate([r[...] for r in aA], axis=1)
         + jnp.concatenate([r[...] for r in aB], axis=1))
    mean = a * inv[...][:, 0:1]
    x = jnp.concatenate([r[...] for r in xs], axis=1)
    y = _bdot(mean, lw[...]) + _bdot(x, rw[...]) + lb[...]
    rows = m * BM + lax.broadcasted_iota(jnp.int32, (BM, 1), 0)
    maskf = (rows < N_NODES).astype(F32)
    y = jnp.where(maskf > 0, y, 0.0)
    nrm = jnp.sqrt(jnp.sum(y * y, axis=1, keepdims=True))
    z = y / jnp.maximum(nrm, 1e-12)
    z = jnp.where(z >= 0, z, 0.01 * z)
    for i, o in enumerate(outs):
        o[...] = z[:, i * PW:(i + 1) * PW]


def _run_sage_mid(aA, aB, xs, inv128, lw, rw, lb, din):
    npan_in = len(aA)
    gh = lw.shape[1]
    npan_out = gh // PW
    body = functools.partial(_sage_mid_body, din)
    pan_spec = pl.BlockSpec((BM, PW), lambda m: (m, 0))
    return pl.pallas_call(
        body,
        grid=(GRID_M,),
        in_specs=(
            [pan_spec] * (3 * npan_in)
            + [
                pl.BlockSpec((BM, 128), lambda m: (m, 0)),
                pl.BlockSpec((din, gh), lambda m: (0, 0)),
                pl.BlockSpec((din, gh), lambda m: (0, 0)),
                pl.BlockSpec((1, gh), lambda m: (0, 0)),
            ]
        ),
        out_specs=[pan_spec] * npan_out,
        out_shape=[jax.ShapeDtypeStruct((M, PW), F32)] * npan_out,
    )(*aA, *aB, *xs, inv128, lw, rw, lb)


def _sage_last_body(din, *refs):
    npan_in = (len(refs) - 9) // 3
    aA = refs[:npan_in]
    aB = refs[npan_in:2 * npan_in]
    xs = refs[2 * npan_in:3 * npan_in]
    inv, lw, rw, lb, clsw, sep, prev = refs[3 * npan_in:3 * npan_in + 7]
    out = refs[3 * npan_in + 7]
    acc = refs[-1]
    m = pl.program_id(0)
    a = (jnp.concatenate([r[...] for r in aA], axis=1)
         + jnp.concatenate([r[...] for r in aB], axis=1))
    mean = a * inv[...][:, 0:1]
    x = jnp.concatenate([r[...] for r in xs], axis=1)
    y = _bdot(mean, lw[...]) + _bdot(x, rw[...]) + lb[...]
    rows = m * BM + lax.broadcasted_iota(jnp.int32, (BM, 1), 0)
    maskf = (rows < N_NODES).astype(F32)
    y = jnp.where(maskf > 0, y, 0.0)
    rowv = m * BM + lax.broadcasted_iota(jnp.int32, (16, BM), 1)
    s = sep[...][:, 0:1]
    p = prev[...][:, 0:1]
    ind = ((rowv >= p) & (rowv < s)).astype(F32)
    contrib = jnp.dot(ind, y, preferred_element_type=F32,
                      precision=lax.Precision.HIGHEST)

    @pl.when(m == 0)
    def _():
        acc[...] = jnp.zeros_like(acc)

    acc[...] += contrib

    @pl.when(m == GRID_M - 1)
    def _():
        out[...] = _bdot(acc[...], clsw[...])


def _run_sage_last(aA, aB, xs, inv128, lw, rw, lb, cls_tiled, sep128, prev128,
                   din):
    npan_in = len(aA)
    gh = lw.shape[1]
    body = functools.partial(_sage_last_body, din)
    pan_spec = pl.BlockSpec((BM, PW), lambda m: (m, 0))
    return pl.pallas_call(
        body,
        grid=(GRID_M,),
        in_specs=(
            [pan_spec] * (3 * npan_in)
            + [
                pl.BlockSpec((BM, 128), lambda m: (m, 0)),
                pl.BlockSpec((din, gh), lambda m: (0, 0)),
                pl.BlockSpec((din, gh), lambda m: (0, 0)),
                pl.BlockSpec((1, gh), lambda m: (0, 0)),
                pl.BlockSpec((gh, 128), lambda m: (0, 0)),
                pl.BlockSpec((16, 128), lambda m: (0, 0)),
                pl.BlockSpec((16, 128), lambda m: (0, 0)),
            ]
        ),
        out_specs=pl.BlockSpec((16, 128), lambda m: (0, 0)),
        out_shape=jax.ShapeDtypeStruct((16, 128), F32),
        scratch_shapes=[pltpu.VMEM((16, gh), F32)],
    )(*aA, *aB, *xs, inv128, lw, rw, lb, cls_tiled, sep128, prev128)


NC, NS = 2, 16
NW = NC * NS
K = 128
RA = 10112
FLUSH = RA // NS


def _sc_mesh():
    return plsc.VectorSubcoreMesh(core_axis_name="c", subcore_axis_name="s",
                                  num_cores=NC, num_subcores=NS)


@functools.partial(pl.kernel, mesh=_sc_mesh(),
                   out_type=jax.ShapeDtypeStruct((M, 128), F32),
                   scratch_types=[
                       pltpu.VMEM((2, K), jnp.int32),
                       pltpu.VMEM((2, K, 128), F32),
                       pltpu.SemaphoreType.DMA,
                   ])
def _embed_gather_sc(ops_hbm, emb_hbm, out_hbm, oidx, rows, sem):
    c = lax.axis_index("c")
    s = lax.axis_index("s")
    wid = s * NC + c
    nch = M // K
    nmine = (nch // NW) + jnp.where(wid < (nch % NW), 1, 0)

    def body(i, _):
        base = (wid + i * NW) * K
        pltpu.sync_copy(ops_hbm.at[pl.ds(base, K)], oidx.at[0])
        pltpu.async_copy(emb_hbm.at[oidx.at[0]], rows.at[0], sem).wait()
        pltpu.sync_copy(rows.at[0], out_hbm.at[pl.ds(base, K)])
        return 0

    lax.fori_loop(0, nmine, body, 0)


def _make_agg(npan, with_count):
    nout = 2 * (npan + (1 if with_count else 0))

    @functools.partial(
        pl.kernel, mesh=_sc_mesh(),
        out_type=[jax.ShapeDtypeStruct((M, PW), F32) for _ in range(nout)],
        scratch_types=[
            pltpu.VMEM((2, K), jnp.int32),
            pltpu.VMEM((2, K), jnp.int32),
            pltpu.VMEM((2, K, PW), F32),
            pltpu.VMEM((K, PW), F32),
            pltpu.VMEM_SHARED((RA, PW), F32),
            pltpu.SemaphoreType.DMA,
        ])
    def agg(src_hbm, dst_hbm, zeros_hbm, *rest):
        nin = npan + (1 if with_count else 0)
        xps = rest[:nin]
        outs = rest[nin:nin + nout]
        sidx, didx, rows, zbuf, accum, sem = rest[nin + nout:]
        c = lax.axis_index("c")
        s = lax.axis_index("s")
        wid = s * NC + c
        nch = E_EDGES // K
        nmine = (nch // NW) + jnp.where(wid < (nch % NW), 1, 0)
        pltpu.sync_copy(zeros_hbm, zbuf)
        if with_count:
            pltpu.sync_copy(xps[npan].at[pl.ds(0, K)], rows.at[1])

        def zero_accum():
            for r in range(FLUSH // K):
                pltpu.sync_copy(zbuf, accum.at[pl.ds(s * FLUSH + r * K, K)])
            rem = FLUSH % K
            if rem:
                pltpu.sync_copy(
                    zbuf.at[pl.ds(0, rem)],
                    accum.at[pl.ds(s * FLUSH + (FLUSH // K) * K, rem)])

        def flush(q):
            for ci in range(NC):
                @pl.when(c == ci)
                def _(q=q, ci=ci):
                    pltpu.sync_copy(
                        accum.at[pl.ds(s * FLUSH, FLUSH)],
                        outs[ci + 2 * q].at[pl.ds(s * FLUSH, FLUSH)])

                @pl.when((c == ci) & (s == NS - 1))
                def _(q=q, ci=ci):
                    pltpu.sync_copy(zbuf, outs[ci + 2 * q].at[pl.ds(RA, M - RA)])

        if with_count:
            zero_accum()
            plsc.subcore_barrier()

            def cbody(i, _):
                base = (wid + i * NW) * K
                pltpu.sync_copy(dst_hbm.at[pl.ds(base, K)], didx.at[0])
                pltpu.sync_copy(rows.at[1], accum.at[didx.at[0]], add=True)
                return 0

            lax.fori_loop(0, nmine, cbody, 0)
            plsc.subcore_barrier()
            flush(npan)

        for p in range(npan):
            zero_accum()
            plsc.subcore_barrier()

            def body(i, _, p=p):
                base = (wid + i * NW) * K
                pltpu.sync_copy(src_hbm.at[pl.ds(base, K)], sidx.at[0])
                pltpu.sync_copy(dst_hbm.at[pl.ds(base, K)], didx.at[0])
                pltpu.async_copy(xps[p].at[sidx.at[0]], rows.at[0], sem).wait()
                pltpu.sync_copy(rows.at[0], accum.at[didx.at[0]], add=True)
                return 0

            lax.fori_loop(0, nmine, body, 0)
            plsc.subcore_barrier()
            flush(p)

    return agg


_AGG = {(2, True): _make_agg(2, True), (4, False): _make_agg(4, False)}


def _aggregate(x_panels, src, dst, with_count=False):
    npan = len(x_panels)
    zeros = jnp.zeros((K, PW), F32)
    args = list(x_panels)
    if with_count:
        ones_col = jnp.zeros((K, PW), F32).at[:, 0].set(1.0)
        args.append(ones_col)
    outs = _AGG[(npan, with_count)](src, dst, zeros, *args)
    aA = [outs[2 * p] for p in range(npan)]
    aB = [outs[2 * p + 1] for p in range(npan)]
    if with_count:
        return aA, aB, outs[2 * npan] + outs[2 * npan + 1]
    return aA, aB


def kernel(node_features, node_separation, node_ops, edges, batches, opcode_emb,
           mlp_W1, mlp_b1, mlp_W2, mlp_b2,
           s0_lW, s0_lb, s0_rW, s1_lW, s1_lb, s1_rW, s2_lW, s2_lb, s2_rW,
           cls_W, cls_b):
    pad = M - N_NODES
    nf_pad = jnp.pad(node_features, ((0, pad), (0, 0)))
    ops_pad = jnp.pad(node_ops, (0, pad))
    src = edges[0]
    dst = edges[1]

    emb128 = jnp.pad(opcode_emb, ((0, 0), (0, 128 - 64)))
    emb_g = _embed_gather_sc(ops_pad, emb128)

    w1a, w1b = mlp_W1[:126], mlp_W1[126:]
    x0p = _run_mlp(nf_pad, emb_g, w1a, w1b, mlp_b1.reshape(1, -1),
                   mlp_W2, mlp_b2.reshape(1, -1))

    aA, aB, cntp = _aggregate(x0p, src, dst, with_count=True)
    cnt = cntp[:, 0:1]
    inv128 = jnp.broadcast_to(1.0 / jnp.maximum(cnt, 1.0), (M, 128))
    x1p = _run_sage_mid(aA, aB, x0p, inv128, s0_lW, s0_rW,
                        s0_lb.reshape(1, -1), din=256)
    aA, aB = _aggregate(x1p, src, dst)
    x2p = _run_sage_mid(aA, aB, x1p, inv128, s1_lW, s1_rW,
                        s1_lb.reshape(1, -1), din=512)
    aA, aB = _aggregate(x2p, src, dst)
    sep128 = jnp.broadcast_to(node_separation.reshape(16, 1), (16, 128))
    prev = jnp.concatenate([jnp.zeros((1,), node_separation.dtype),
                            node_separation[:15]])
    prev128 = jnp.broadcast_to(prev.reshape(16, 1), (16, 128))
    cls_tiled = jnp.broadcast_to(cls_W, (cls_W.shape[0], 128))
    out128 = _run_sage_last(aA, aB, x2p, inv128, s2_lW, s2_rW,
                            s2_lb.reshape(1, -1), cls_tiled, sep128, prev128,
                            din=512)

    t = out128[:, 0]
    return jnp.zeros((16,), F32).at[batches].set(t) + cls_b
